# baseline (speedup 1.0000x reference)
import functools

import jax
import jax.numpy as jnp
from jax import lax
from jax.experimental import pallas as pl
from jax.experimental.pallas import tpu as pltpu
from jax.experimental.pallas import tpu_sc as plsc

EMBED = 32
PACK = 128 // EMBED
CH = 320
NBUF = 2


@functools.lru_cache(maxsize=None)
def _make_gather(n_idx, vocab128):
    info = plsc.get_sparse_core_info()
    nc, ns, nl = info.num_cores, info.num_subcores, info.num_lanes
    nw = nc * ns
    b_per_w = n_idx // nw
    n_chunk = b_per_w // CH
    assert n_chunk * CH * nw == n_idx and CH % nl == 0 and n_chunk % NBUF == 0
    n_grp = CH // nl
    mesh = plsc.VectorSubcoreMesh(core_axis_name="c", subcore_axis_name="s")

    @functools.partial(
        pl.kernel,
        mesh=mesh,
        compiler_params=pltpu.CompilerParams(
            use_tc_tiling_on_sc=False, needs_layout_passes=False
        ),
        out_type=jax.ShapeDtypeStruct((EMBED, n_idx), jnp.float32),
        scratch_types=[
            pltpu.VMEM((n_chunk, CH), jnp.int32),
            pltpu.VMEM((n_chunk, CH), jnp.int32),
            pltpu.VMEM((NBUF, CH, 128), jnp.float32),
            pltpu.VMEM((NBUF, EMBED, CH), jnp.float32),
            pltpu.SemaphoreType.DMA((NBUF,)),
            pltpu.SemaphoreType.DMA((NBUF,)),
        ],
    )
    def k(slot_hbm, off_hbm, table_hbm, out_hbm, slot_v, off_v, rows_v, st_v,
          gsem, osem):
        wid = lax.axis_index("s") * nc + lax.axis_index("c")
        base = wid * b_per_w
        pltpu.sync_copy(slot_hbm.at[wid], slot_v)
        pltpu.sync_copy(off_hbm.at[wid], off_v)

        def fire(c, i):
            pltpu.async_copy(table_hbm.at[slot_v.at[c]], rows_v.at[i], gsem.at[i])

        def extract(c, i):
            rows = rows_v.at[i]
            st = st_v.at[i]

            def g_body(g, carry):
                jvec = lax.iota(jnp.int32, nl) + g * nl
                offs = off_v[c, pl.ds(g * nl, nl)]
                for e in range(EMBED):
                    st[e, pl.ds(g * nl, nl)] = plsc.load_gather(
                        rows, [jvec, offs + e]
                    )
                return carry

            lax.fori_loop(0, n_grp, g_body, 0)

        for i in range(NBUF):
            fire(i, i)

        def chunk_body(m, carry):
            for i in range(NBUF):
                c = m * NBUF + i
                pltpu.make_async_copy(
                    table_hbm.at[slot_v.at[c]], rows_v.at[i], gsem.at[i]
                ).wait()

                @pl.when(c >= NBUF)
                def _():
                    pltpu.make_async_copy(
                        st_v.at[i],
                        out_hbm.at[:, pl.ds(base + (c - NBUF) * CH, CH)],
                        osem.at[i],
                    ).wait()

                extract(c, i)
                pltpu.async_copy(
                    st_v.at[i],
                    out_hbm.at[:, pl.ds(base + c * CH, CH)],
                    osem.at[i],
                )

                @pl.when(c + NBUF < n_chunk)
                def _():
                    fire(c + NBUF, i)

            return carry

        lax.fori_loop(0, n_chunk // NBUF, chunk_body, 0)

        for c in range(n_chunk - NBUF, n_chunk):
            i = c % NBUF
            pltpu.make_async_copy(
                st_v.at[i], out_hbm.at[:, pl.ds(base + c * CH, CH)], osem.at[i]
            ).wait()

    return k


def kernel(inputs, embeddings):
    b, l = inputs.shape
    n = b * l
    info = plsc.get_sparse_core_info()
    nw = info.num_cores * info.num_subcores
    n_chunk = n // nw // CH
    flat = inputs.reshape(nw, n_chunk, CH).astype(jnp.int32)
    slots = flat // PACK
    offs = (flat % PACK) * EMBED
    vocab, embed = embeddings.shape
    table128 = embeddings.reshape(vocab // PACK, 128)
    out_t = _make_gather(n, vocab // PACK)(slots, offs, table128)
    return jnp.swapaxes(out_t, 0, 1).reshape(b, l, embed)


# speedup vs baseline: 1.3864x; 1.3864x over previous
import functools

import jax
import jax.numpy as jnp
from jax import lax
from jax.experimental import pallas as pl
from jax.experimental.pallas import tpu as pltpu
from jax.experimental.pallas import tpu_sc as plsc

EMBED = 32
PACK = 128 // EMBED
CH = 320
NBUF = 2
CB = 16384
SB = CB // PACK


@functools.lru_cache(maxsize=None)
def _make_tc_transpose(vocab):
    n_blk = (vocab + CB - 1) // CB

    def body(x_ref, o_ref):
        x = x_ref[...]
        o_ref[...] = jnp.concatenate(
            [
                jnp.swapaxes(x[:, j * SB:(j + 1) * SB], 0, 1)
                for j in range(PACK)
            ],
            axis=1,
        )

    return pl.pallas_call(
        body,
        grid=(n_blk,),
        in_specs=[pl.BlockSpec((EMBED, CB), lambda i: (0, i))],
        out_specs=pl.BlockSpec((SB, 128), lambda i: (i, 0)),
        out_shape=jax.ShapeDtypeStruct((n_blk * SB, 128), jnp.float32),
    )


@functools.lru_cache(maxsize=None)
def _make_gather(n_idx, vocab128):
    info = plsc.get_sparse_core_info()
    nc, ns, nl = info.num_cores, info.num_subcores, info.num_lanes
    nw = nc * ns
    b_per_w = n_idx // nw
    n_chunk = b_per_w // CH
    assert n_chunk * CH * nw == n_idx and CH % nl == 0 and n_chunk % NBUF == 0
    n_grp = CH // nl
    mesh = plsc.VectorSubcoreMesh(core_axis_name="c", subcore_axis_name="s")

    @functools.partial(
        pl.kernel,
        mesh=mesh,
        compiler_params=pltpu.CompilerParams(
            use_tc_tiling_on_sc=False, needs_layout_passes=False
        ),
        out_type=jax.ShapeDtypeStruct((EMBED, n_idx), jnp.float32),
        scratch_types=[
            pltpu.VMEM((n_chunk, CH), jnp.int32),
            pltpu.VMEM((n_chunk, CH), jnp.int32),
            pltpu.VMEM((NBUF, CH, 128), jnp.float32),
            pltpu.VMEM((NBUF, EMBED, CH), jnp.float32),
            pltpu.SemaphoreType.DMA((NBUF,)),
            pltpu.SemaphoreType.DMA((NBUF,)),
        ],
    )
    def k(slot_hbm, off_hbm, table_hbm, out_hbm, slot_v, off_v, rows_v, st_v,
          gsem, osem):
        wid = lax.axis_index("s") * nc + lax.axis_index("c")
        base = wid * b_per_w
        pltpu.sync_copy(slot_hbm.at[wid], slot_v)
        pltpu.sync_copy(off_hbm.at[wid], off_v)

        def fire(c, i):
            pltpu.async_copy(table_hbm.at[slot_v.at[c]], rows_v.at[i], gsem.at[i])

        def extract(c, i):
            rows = rows_v.at[i]
            st = st_v.at[i]

            def g_body(g, carry):
                jvec = lax.iota(jnp.int32, nl) + g * nl
                offs = off_v[c, pl.ds(g * nl, nl)]
                for e in range(EMBED):
                    st[e, pl.ds(g * nl, nl)] = plsc.load_gather(
                        rows, [jvec, offs + e]
                    )
                return carry

            lax.fori_loop(0, n_grp, g_body, 0)

        for i in range(NBUF):
            fire(i, i)

        def chunk_body(m, carry):
            for i in range(NBUF):
                c = m * NBUF + i
                pltpu.make_async_copy(
                    table_hbm.at[slot_v.at[c]], rows_v.at[i], gsem.at[i]
                ).wait()

                @pl.when(c >= NBUF)
                def _():
                    pltpu.make_async_copy(
                        st_v.at[i],
                        out_hbm.at[:, pl.ds(base + (c - NBUF) * CH, CH)],
                        osem.at[i],
                    ).wait()

                extract(c, i)
                pltpu.async_copy(
                    st_v.at[i],
                    out_hbm.at[:, pl.ds(base + c * CH, CH)],
                    osem.at[i],
                )

                @pl.when(c + NBUF < n_chunk)
                def _():
                    fire(c + NBUF, i)

            return carry

        lax.fori_loop(0, n_chunk // NBUF, chunk_body, 0)

        for c in range(n_chunk - NBUF, n_chunk):
            i = c % NBUF
            pltpu.make_async_copy(
                st_v.at[i], out_hbm.at[:, pl.ds(base + c * CH, CH)], osem.at[i]
            ).wait()

    return k


def kernel(inputs, embeddings):
    b, l = inputs.shape
    n = b * l
    info = plsc.get_sparse_core_info()
    nw = info.num_cores * info.num_subcores
    n_chunk = n // nw // CH
    flat = inputs.reshape(nw, n_chunk, CH).astype(jnp.int32)
    vocab, embed = embeddings.shape
    slots = (flat // CB) * SB + flat % SB
    offs = ((flat % CB) // SB) * EMBED
    emb_t = jnp.swapaxes(embeddings, 0, 1)
    table128 = _make_tc_transpose(vocab)(emb_t)
    out_t = _make_gather(n, vocab // PACK)(slots, offs, table128)
    return jnp.swapaxes(out_t, 0, 1).reshape(b, l, embed)


# speedup vs baseline: 1.5626x; 1.1271x over previous
import functools

import jax
import jax.numpy as jnp
from jax import lax
from jax.experimental import pallas as pl
from jax.experimental.pallas import tpu as pltpu
from jax.experimental.pallas import tpu_sc as plsc

EMBED = 32
PACK = 128 // EMBED
CH = 320
NBUF = 2
CB = 16384
SB = CB // PACK


@functools.lru_cache(maxsize=None)
def _make_tc_transpose(vocab):
    n_blk = (vocab + CB - 1) // CB

    def body(x_ref, o_ref):
        x = x_ref[...]
        o_ref[...] = jnp.concatenate(
            [
                jnp.swapaxes(x[:, j * SB:(j + 1) * SB], 0, 1)
                for j in range(PACK)
            ],
            axis=1,
        )

    return pl.pallas_call(
        body,
        grid=(n_blk,),
        in_specs=[pl.BlockSpec((EMBED, CB), lambda i: (0, i))],
        out_specs=pl.BlockSpec((SB, 128), lambda i: (i, 0)),
        out_shape=jax.ShapeDtypeStruct((n_blk * SB, 128), jnp.float32),
    )


@functools.lru_cache(maxsize=None)
def _make_gather(n_idx, vocab128):
    info = plsc.get_sparse_core_info()
    nc, ns, nl = info.num_cores, info.num_subcores, info.num_lanes
    nw = nc * ns
    b_per_w = n_idx // nw
    n_chunk = b_per_w // CH
    assert n_chunk * CH * nw == n_idx and CH % nl == 0 and n_chunk % NBUF == 0
    n_grp = CH // nl
    mesh = plsc.VectorSubcoreMesh(core_axis_name="c", subcore_axis_name="s")

    @functools.partial(
        pl.kernel,
        mesh=mesh,
        compiler_params=pltpu.CompilerParams(
            use_tc_tiling_on_sc=False, needs_layout_passes=False
        ),
        out_type=jax.ShapeDtypeStruct((EMBED, n_idx), jnp.float32),
        scratch_types=[
            pltpu.VMEM((n_chunk, CH), jnp.int32),
            pltpu.VMEM((n_chunk, CH), jnp.int32),
            pltpu.VMEM((NBUF, CH, 128), jnp.float32),
            pltpu.VMEM((NBUF, EMBED, CH), jnp.float32),
            pltpu.SemaphoreType.DMA((NBUF,)),
            pltpu.SemaphoreType.DMA((NBUF,)),
        ],
    )
    def k(slot_hbm, off_hbm, table_hbm, out_hbm, slot_v, off_v, rows_v, st_v,
          gsem, osem):
        wid = lax.axis_index("s") * nc + lax.axis_index("c")
        base = wid * b_per_w
        pltpu.sync_copy(slot_hbm.at[wid], slot_v)
        pltpu.sync_copy(off_hbm.at[wid], off_v)

        def fire(c, i):
            pltpu.async_copy(table_hbm.at[slot_v.at[c]], rows_v.at[i], gsem.at[i])

        def extract(c, i):
            rows = rows_v.at[i]
            st = st_v.at[i]

            @plsc.parallel_loop(0, n_grp, unroll=2)
            def g_body(g):
                jvec = lax.iota(jnp.int32, nl) + g * nl
                offs = off_v[c, pl.ds(g * nl, nl)]
                for e in range(EMBED):
                    st[e, pl.ds(g * nl, nl)] = plsc.load_gather(
                        rows, [jvec, offs + e]
                    )

        for i in range(NBUF):
            fire(i, i)

        def chunk_body(m, carry):
            for i in range(NBUF):
                c = m * NBUF + i
                pltpu.make_async_copy(
                    table_hbm.at[slot_v.at[c]], rows_v.at[i], gsem.at[i]
                ).wait()

                @pl.when(c >= NBUF)
                def _():
                    pltpu.make_async_copy(
                        st_v.at[i],
                        out_hbm.at[:, pl.ds(base + (c - NBUF) * CH, CH)],
                        osem.at[i],
                    ).wait()

                extract(c, i)
                pltpu.async_copy(
                    st_v.at[i],
                    out_hbm.at[:, pl.ds(base + c * CH, CH)],
                    osem.at[i],
                )

                @pl.when(c + NBUF < n_chunk)
                def _():
                    fire(c + NBUF, i)

            return carry

        lax.fori_loop(0, n_chunk // NBUF, chunk_body, 0)

        for c in range(n_chunk - NBUF, n_chunk):
            i = c % NBUF
            pltpu.make_async_copy(
                st_v.at[i], out_hbm.at[:, pl.ds(base + c * CH, CH)], osem.at[i]
            ).wait()

    return k


def kernel(inputs, embeddings):
    b, l = inputs.shape
    n = b * l
    info = plsc.get_sparse_core_info()
    nw = info.num_cores * info.num_subcores
    n_chunk = n // nw // CH
    flat = inputs.reshape(nw, n_chunk, CH).astype(jnp.int32)
    vocab, embed = embeddings.shape
    slots = (flat // CB) * SB + flat % SB
    offs = ((flat % CB) // SB) * EMBED
    emb_t = jnp.swapaxes(embeddings, 0, 1)
    table128 = _make_tc_transpose(vocab)(emb_t)
    out_t = _make_gather(n, vocab // PACK)(slots, offs, table128)
    return jnp.swapaxes(out_t, 0, 1).reshape(b, l, embed)
